# Initial kernel scaffold; baseline (speedup 1.0000x reference)
import jax
import jax.numpy as jnp
from jax.experimental import pallas as pl


def kernel(point_cloud, W_vfe, b_vfe, Wqkv, bqkv, Wo, bo, ln1_g, ln1_b, ln2_g, ln2_b, W1, b1, W2, b2, pos_embed, Wq_p, bq_p, Wk_p, bk_p, Wv_p, bv_p, Wc, bc):
    raise NotImplementedError("write your pallas kernel here")


# speedup vs baseline: 1.0737x; 1.0737x over previous
import jax
import jax.numpy as jnp
from jax.experimental import pallas as pl
from jax.experimental.pallas import tpu as pltpu

B, N, CIN, D, W, NH, NB, ED = 2, 16384, 10, 128, 128, 8, 2, 512
DH = D // NH
PDH = ED // NH
T = 1024
NWC = T // W
NCH = (B * N) // T
CPB = N // T
TC = 2048
NCC = N // TC

_F32 = jnp.float32


def _ln(x, g, b):
    m = jnp.mean(x, axis=-1, keepdims=True)
    d = x - m
    v = jnp.mean(d * d, axis=-1, keepdims=True)
    return d * jax.lax.rsqrt(v + 1e-5) * g + b


def _tblock(x, wq, bq, wk, bk, wv, bv, wo, bo, g1, s1, g2, s2, w1, c1, w2, c2):
    q = x @ wq + bq
    k = x @ wk + bk
    v = x @ wv + bv
    outs = []
    for h in range(NH):
        qh = q[:, h * DH:(h + 1) * DH].reshape(NWC, W, DH)
        kh = k[:, h * DH:(h + 1) * DH].reshape(NWC, W, DH)
        vh = v[:, h * DH:(h + 1) * DH].reshape(NWC, W, DH)
        s = jax.lax.dot_general(qh, kh, (((2,), (2,)), ((0,), (0,))),
                                preferred_element_type=_F32) * 0.25
        s = s - jnp.max(s, axis=-1, keepdims=True)
        e = jnp.exp(s)
        p = e / jnp.sum(e, axis=-1, keepdims=True)
        oh = jax.lax.dot_general(p, vh, (((2,), (1,)), ((0,), (0,))),
                                 preferred_element_type=_F32)
        outs.append(oh.reshape(T, DH))
    o = jnp.concatenate(outs, axis=1) @ wo + bo
    x = _ln(x + o, g1, s1)
    f = jnp.maximum(x @ w1 + c1, 0.0) @ w2 + c2
    return _ln(x + f, g2, s2)


def _stage1_body(pc_ref, wvfe_ref, bvfe_ref,
                 wq_ref, bq_ref, wk_ref, bk_ref, wv_ref, bv_ref, wo_ref, bo_ref,
                 g1_ref, s1_ref, g2_ref, s2_ref, w1_ref, c1_ref, w2_ref, c2_ref,
                 out_ref):
    x = jnp.maximum(pc_ref[...] @ wvfe_ref[...] + bvfe_ref[...], 0.0)
    out_ref[...] = _tblock(
        x, wq_ref[...], bq_ref[...], wk_ref[...], bk_ref[...], wv_ref[...],
        bv_ref[...], wo_ref[...], bo_ref[...], g1_ref[...], s1_ref[...],
        g2_ref[...], s2_ref[...], w1_ref[...], c1_ref[...], w2_ref[...],
        c2_ref[...])


def _stage2_body(x_ref,
                 wq_ref, bq_ref, wk_ref, bk_ref, wv_ref, bv_ref, wo_ref, bo_ref,
                 g1_ref, s1_ref, g2_ref, s2_ref, w1_ref, c1_ref, w2_ref, c2_ref,
                 out_ref, sum_ref):
    c = pl.program_id(0)
    y = _tblock(
        x_ref[...], wq_ref[...], bq_ref[...], wk_ref[...], bk_ref[...],
        wv_ref[...], bv_ref[...], wo_ref[...], bo_ref[...], g1_ref[...],
        s1_ref[...], g2_ref[...], s2_ref[...], w1_ref[...], c1_ref[...],
        w2_ref[...], c2_ref[...])
    out_ref[...] = y

    @pl.when(c % CPB == 0)
    def _():
        sum_ref[...] = jnp.zeros_like(sum_ref)

    sum_ref[...] += jnp.sum(y, axis=0, keepdims=True)[None]


def _stage3_body(x_ref, pe_ref, q_ref, s0_ref, v0_ref,
                 wk_ref, bk_ref, wv_ref, bv_ref, wc_ref, bc_ref,
                 out_ref, m_ref, l_ref, acc_ref):
    c = pl.program_id(1)

    @pl.when(c == 0)
    def _():
        m_ref[...] = s0_ref[0]
        l_ref[...] = jnp.ones_like(l_ref)
        acc_ref[...] = v0_ref[0]

    x = x_ref[0] + pe_ref[...]
    k = x @ wk_ref[...] + bk_ref[...]
    v = x @ wv_ref[...] + bv_ref[...]
    q8 = q_ref[0]
    for h in range(NH):
        qh = q8[h:h + 1, :]
        kh = k[:, h * PDH:(h + 1) * PDH]
        vh = v[:, h * PDH:(h + 1) * PDH]
        sh = jax.lax.dot_general(qh, kh, (((1,), (1,)), ((), ())),
                                 preferred_element_type=_F32) * 0.125
        m_old = m_ref[h:h + 1, 0:1]
        m_new = jnp.maximum(m_old, jnp.max(sh, axis=1, keepdims=True))
        corr = jnp.exp(m_old - m_new)
        p = jnp.exp(sh - m_new)
        l_ref[h:h + 1, :] = jnp.broadcast_to(
            l_ref[h:h + 1, 0:1] * corr + jnp.sum(p, axis=1, keepdims=True),
            (1, 128))
        acc_ref[h:h + 1, :] = acc_ref[h:h + 1, :] * corr + jax.lax.dot_general(
            p, vh, (((1,), (0,)), ((), ())), preferred_element_type=_F32)
        m_ref[h:h + 1, :] = jnp.broadcast_to(m_new, (1, 128))

    @pl.when(c == NCC - 1)
    def _():
        out = bc_ref[...]
        for h in range(NH):
            oh = acc_ref[h:h + 1, :] / l_ref[h:h + 1, 0:1]
            out = out + jax.lax.dot_general(
                oh, wc_ref[h * PDH:(h + 1) * PDH, :], (((1,), (0,)), ((), ())),
                preferred_element_type=_F32)
        out_ref[...] = out[None]


def _full(shape):
    n = len(shape)
    return pl.BlockSpec(shape, lambda *_: (0,) * n)


def kernel(point_cloud, W_vfe, b_vfe, Wqkv, bqkv, Wo, bo, ln1_g, ln1_b,
           ln2_g, ln2_b, W1, b1, W2, b2, pos_embed, Wq_p, bq_p, Wk_p, bk_p,
           Wv_p, bv_p, Wc, bc):
    f32 = _F32
    pc = jnp.pad(point_cloud.reshape(B * N, CIN), ((0, 0), (0, D - CIN)))
    wvfe = jnp.pad(W_vfe, ((0, D - CIN), (0, 0)))

    def blk_weights(i):
        return (Wqkv[i, 0], bqkv[i, 0].reshape(1, D),
                Wqkv[i, 1], bqkv[i, 1].reshape(1, D),
                Wqkv[i, 2], bqkv[i, 2].reshape(1, D),
                Wo[i], bo[i].reshape(1, D),
                ln1_g[i].reshape(1, D), ln1_b[i].reshape(1, D),
                ln2_g[i].reshape(1, D), ln2_b[i].reshape(1, D),
                W1[i], b1[i].reshape(1, 4 * D),
                W2[i], b2[i].reshape(1, D))

    wspecs = [_full((D, D)), _full((1, D)), _full((D, D)), _full((1, D)),
              _full((D, D)), _full((1, D)), _full((D, D)), _full((1, D)),
              _full((1, D)), _full((1, D)), _full((1, D)), _full((1, D)),
              _full((D, 4 * D)), _full((1, 4 * D)),
              _full((4 * D, D)), _full((1, D))]

    x1 = pl.pallas_call(
        _stage1_body,
        grid=(NCH,),
        in_specs=[pl.BlockSpec((T, D), lambda c: (c, 0)),
                  _full((D, D)), _full((1, D))] + wspecs,
        out_specs=pl.BlockSpec((T, D), lambda c: (c, 0)),
        out_shape=jax.ShapeDtypeStruct((B * N, D), f32),
    )(pc, wvfe, b_vfe.reshape(1, D), *blk_weights(0))

    x1r = jnp.roll(x1.reshape(B, N, D), W // 2, axis=1).reshape(B * N, D)

    x2, sums = pl.pallas_call(
        _stage2_body,
        grid=(NCH,),
        in_specs=[pl.BlockSpec((T, D), lambda c: (c, 0))] + wspecs,
        out_specs=[pl.BlockSpec((T, D), lambda c: (c, 0)),
                   pl.BlockSpec((1, 1, D), lambda c: (c // CPB, 0, 0))],
        out_shape=[jax.ShapeDtypeStruct((B * N, D), f32),
                   jax.ShapeDtypeStruct((B, 1, D), f32)],
    )(x1r, *blk_weights(1))

    mean = sums[:, 0, :] / N
    tok0 = mean + pos_embed[0][None, :]
    q_p = (tok0 @ Wq_p + bq_p).reshape(B, NH, PDH)
    k0 = (tok0 @ Wk_p + bk_p).reshape(B, NH, PDH)
    v0 = (tok0 @ Wv_p + bv_p).reshape(B, NH, PDH)
    s0 = jnp.sum(q_p * k0, axis=-1) * 0.125
    s0b = jnp.broadcast_to(s0[:, :, None], (B, NH, 128))

    out = pl.pallas_call(
        _stage3_body,
        grid=(B, NCC),
        in_specs=[pl.BlockSpec((1, TC, D), lambda b, c: (b, c, 0)),
                  pl.BlockSpec((TC, D), lambda b, c: (c, 0)),
                  pl.BlockSpec((1, NH, PDH), lambda b, c: (b, 0, 0)),
                  pl.BlockSpec((1, NH, 128), lambda b, c: (b, 0, 0)),
                  pl.BlockSpec((1, NH, PDH), lambda b, c: (b, 0, 0)),
                  _full((D, ED)), _full((1, ED)),
                  _full((D, ED)), _full((1, ED)),
                  _full((ED, ED)), _full((1, ED))],
        out_specs=pl.BlockSpec((1, 1, ED), lambda b, c: (b, 0, 0)),
        out_shape=jax.ShapeDtypeStruct((B, 1, ED), f32),
        scratch_shapes=[pltpu.VMEM((NH, 128), f32),
                        pltpu.VMEM((NH, 128), f32),
                        pltpu.VMEM((NH, PDH), f32)],
    )(x2.reshape(B, N, D), pos_embed[1:], q_p, s0b, v0,
      Wk_p, bk_p.reshape(1, ED), Wv_p, bv_p.reshape(1, ED),
      Wc, bc.reshape(1, ED))

    return out[:, 0, :]


# speedup vs baseline: 1.0880x; 1.0133x over previous
import jax
import jax.numpy as jnp
from jax.experimental import pallas as pl
from jax.experimental.pallas import tpu as pltpu

B, N, CIN, D, W, NH, NB, ED = 2, 16384, 10, 128, 128, 8, 2, 512
DH = D // NH
PDH = ED // NH
T = 1024
NWC = T // W
NCH = (B * N) // T
CPB = N // T
TC = 2048
NCC = N // TC

_F32 = jnp.float32
_BF16 = jnp.bfloat16


def _ln(x, g, b):
    m = jnp.mean(x, axis=-1, keepdims=True)
    d = x - m
    v = jnp.mean(d * d, axis=-1, keepdims=True)
    return d * jax.lax.rsqrt(v + 1e-5) * g + b


def _bdot(a, b, dims):
    return jax.lax.dot_general(a.astype(_BF16), b.astype(_BF16), dims,
                               preferred_element_type=_F32)


def _bmm(a, w):
    return jax.lax.dot_general(a.astype(_BF16), w, (((1,), (0,)), ((), ())),
                               preferred_element_type=_F32)


def _tblock(x, wq, bq, wk, bk, wv, bv, wo, bo, g1, s1, g2, s2, w1, c1, w2, c2):
    q = _bmm(x, wq) + bq
    k = _bmm(x, wk) + bk
    v = _bmm(x, wv) + bv
    outs = []
    for h in range(NH):
        qh = q[:, h * DH:(h + 1) * DH].reshape(NWC, W, DH)
        kh = k[:, h * DH:(h + 1) * DH].reshape(NWC, W, DH)
        vh = v[:, h * DH:(h + 1) * DH].reshape(NWC, W, DH)
        s = _bdot(qh, kh, (((2,), (2,)), ((0,), (0,)))) * 0.25
        s = s - jnp.max(s, axis=-1, keepdims=True)
        e = jnp.exp(s)
        p = e / jnp.sum(e, axis=-1, keepdims=True)
        oh = _bdot(p, vh, (((2,), (1,)), ((0,), (0,))))
        outs.append(oh.reshape(T, DH))
    o = _bmm(jnp.concatenate(outs, axis=1), wo) + bo
    x = _ln(x + o, g1, s1)
    f = _bmm(jnp.maximum(_bmm(x, w1) + c1, 0.0), w2) + c2
    return _ln(x + f, g2, s2)


def _stage1_body(pc_ref, wvfe_ref, bvfe_ref,
                 wq_ref, bq_ref, wk_ref, bk_ref, wv_ref, bv_ref, wo_ref, bo_ref,
                 g1_ref, s1_ref, g2_ref, s2_ref, w1_ref, c1_ref, w2_ref, c2_ref,
                 out_ref):
    x = jnp.maximum(_bmm(pc_ref[...], wvfe_ref[...]) + bvfe_ref[...], 0.0)
    out_ref[...] = _tblock(
        x, wq_ref[...], bq_ref[...], wk_ref[...], bk_ref[...], wv_ref[...],
        bv_ref[...], wo_ref[...], bo_ref[...], g1_ref[...], s1_ref[...],
        g2_ref[...], s2_ref[...], w1_ref[...], c1_ref[...], w2_ref[...],
        c2_ref[...])


def _stage2_body(x_ref,
                 wq_ref, bq_ref, wk_ref, bk_ref, wv_ref, bv_ref, wo_ref, bo_ref,
                 g1_ref, s1_ref, g2_ref, s2_ref, w1_ref, c1_ref, w2_ref, c2_ref,
                 out_ref, sum_ref):
    c = pl.program_id(0)
    y = _tblock(
        x_ref[...], wq_ref[...], bq_ref[...], wk_ref[...], bk_ref[...],
        wv_ref[...], bv_ref[...], wo_ref[...], bo_ref[...], g1_ref[...],
        s1_ref[...], g2_ref[...], s2_ref[...], w1_ref[...], c1_ref[...],
        w2_ref[...], c2_ref[...])
    out_ref[...] = y

    @pl.when(c % CPB == 0)
    def _():
        sum_ref[...] = jnp.zeros_like(sum_ref)

    sum_ref[...] += jnp.sum(y, axis=0, keepdims=True)[None]


def _stage3_body(x_ref, pe_ref, q_ref, s0_ref, v0_ref,
                 wk_ref, bk_ref, wv_ref, bv_ref, wc_ref, bc_ref,
                 out_ref, m_ref, l_ref, acc_ref):
    c = pl.program_id(1)

    @pl.when(c == 0)
    def _():
        m_ref[...] = s0_ref[0]
        l_ref[...] = jnp.ones_like(l_ref)
        acc_ref[...] = v0_ref[0]

    x = x_ref[0] + pe_ref[...]
    k = _bmm(x, wk_ref[...]) + bk_ref[...]
    v = _bmm(x, wv_ref[...]) + bv_ref[...]
    q8 = q_ref[0]
    for h in range(NH):
        qh = q8[h:h + 1, :]
        kh = k[:, h * PDH:(h + 1) * PDH]
        vh = v[:, h * PDH:(h + 1) * PDH]
        sh = _bdot(qh, kh, (((1,), (1,)), ((), ()))) * 0.125
        m_old = m_ref[h:h + 1, 0:1]
        m_new = jnp.maximum(m_old, jnp.max(sh, axis=1, keepdims=True))
        corr = jnp.exp(m_old - m_new)
        p = jnp.exp(sh - m_new)
        l_ref[h:h + 1, :] = jnp.broadcast_to(
            l_ref[h:h + 1, 0:1] * corr + jnp.sum(p, axis=1, keepdims=True),
            (1, 128))
        acc_ref[h:h + 1, :] = acc_ref[h:h + 1, :] * corr + _bdot(
            p, vh, (((1,), (0,)), ((), ())))
        m_ref[h:h + 1, :] = jnp.broadcast_to(m_new, (1, 128))

    @pl.when(c == NCC - 1)
    def _():
        out = bc_ref[...]
        for h in range(NH):
            oh = acc_ref[h:h + 1, :] / l_ref[h:h + 1, 0:1]
            out = out + _bdot(oh, wc_ref[h * PDH:(h + 1) * PDH, :],
                              (((1,), (0,)), ((), ())))
        out_ref[...] = out[None]


def _full(shape):
    n = len(shape)
    return pl.BlockSpec(shape, lambda *_: (0,) * n)


def kernel(point_cloud, W_vfe, b_vfe, Wqkv, bqkv, Wo, bo, ln1_g, ln1_b,
           ln2_g, ln2_b, W1, b1, W2, b2, pos_embed, Wq_p, bq_p, Wk_p, bk_p,
           Wv_p, bv_p, Wc, bc):
    f32 = _F32
    bf16 = _BF16
    pc = jnp.pad(point_cloud.reshape(B * N, CIN),
                 ((0, 0), (0, D - CIN))).astype(bf16)
    wvfe = jnp.pad(W_vfe, ((0, D - CIN), (0, 0))).astype(bf16)

    def blk_weights(i):
        return (Wqkv[i, 0].astype(bf16), bqkv[i, 0].reshape(1, D),
                Wqkv[i, 1].astype(bf16), bqkv[i, 1].reshape(1, D),
                Wqkv[i, 2].astype(bf16), bqkv[i, 2].reshape(1, D),
                Wo[i].astype(bf16), bo[i].reshape(1, D),
                ln1_g[i].reshape(1, D), ln1_b[i].reshape(1, D),
                ln2_g[i].reshape(1, D), ln2_b[i].reshape(1, D),
                W1[i].astype(bf16), b1[i].reshape(1, 4 * D),
                W2[i].astype(bf16), b2[i].reshape(1, D))

    wspecs = [_full((D, D)), _full((1, D)), _full((D, D)), _full((1, D)),
              _full((D, D)), _full((1, D)), _full((D, D)), _full((1, D)),
              _full((1, D)), _full((1, D)), _full((1, D)), _full((1, D)),
              _full((D, 4 * D)), _full((1, 4 * D)),
              _full((4 * D, D)), _full((1, D))]

    x1 = pl.pallas_call(
        _stage1_body,
        grid=(NCH,),
        in_specs=[pl.BlockSpec((T, D), lambda c: (c, 0)),
                  _full((D, D)), _full((1, D))] + wspecs,
        out_specs=pl.BlockSpec((T, D), lambda c: (c, 0)),
        out_shape=jax.ShapeDtypeStruct((B * N, D), f32),
    )(pc, wvfe, b_vfe.reshape(1, D), *blk_weights(0))

    x1r = jnp.roll(x1.reshape(B, N, D), W // 2, axis=1).reshape(B * N, D)

    x2, sums = pl.pallas_call(
        _stage2_body,
        grid=(NCH,),
        in_specs=[pl.BlockSpec((T, D), lambda c: (c, 0))] + wspecs,
        out_specs=[pl.BlockSpec((T, D), lambda c: (c, 0)),
                   pl.BlockSpec((1, 1, D), lambda c: (c // CPB, 0, 0))],
        out_shape=[jax.ShapeDtypeStruct((B * N, D), f32),
                   jax.ShapeDtypeStruct((B, 1, D), f32)],
    )(x1r, *blk_weights(1))

    mean = sums[:, 0, :] / N
    tok0 = mean + pos_embed[0][None, :]
    q_p = (tok0 @ Wq_p + bq_p).reshape(B, NH, PDH)
    k0 = (tok0 @ Wk_p + bk_p).reshape(B, NH, PDH)
    v0 = (tok0 @ Wv_p + bv_p).reshape(B, NH, PDH)
    s0 = jnp.sum(q_p * k0, axis=-1) * 0.125
    s0b = jnp.broadcast_to(s0[:, :, None], (B, NH, 128))

    out = pl.pallas_call(
        _stage3_body,
        grid=(B, NCC),
        in_specs=[pl.BlockSpec((1, TC, D), lambda b, c: (b, c, 0)),
                  pl.BlockSpec((TC, D), lambda b, c: (c, 0)),
                  pl.BlockSpec((1, NH, PDH), lambda b, c: (b, 0, 0)),
                  pl.BlockSpec((1, NH, 128), lambda b, c: (b, 0, 0)),
                  pl.BlockSpec((1, NH, PDH), lambda b, c: (b, 0, 0)),
                  _full((D, ED)), _full((1, ED)),
                  _full((D, ED)), _full((1, ED)),
                  _full((ED, ED)), _full((1, ED))],
        out_specs=pl.BlockSpec((1, 1, ED), lambda b, c: (b, 0, 0)),
        out_shape=jax.ShapeDtypeStruct((B, 1, ED), f32),
        scratch_shapes=[pltpu.VMEM((NH, 128), f32),
                        pltpu.VMEM((NH, 128), f32),
                        pltpu.VMEM((NH, PDH), f32)],
    )(x2.reshape(B, N, D), pos_embed[1:], q_p, s0b, v0,
      Wk_p.astype(bf16), bk_p.reshape(1, ED),
      Wv_p.astype(bf16), bv_p.reshape(1, ED),
      Wc.astype(bf16), bc.reshape(1, ED))

    return out[:, 0, :]
